# baseline (speedup 1.0000x reference)
import functools

import jax
import jax.numpy as jnp
from jax import lax
from jax.experimental import pallas as pl
from jax.experimental.pallas import tpu as pltpu
from jax.experimental.pallas import tpu_sc as plsc

M = 20000
B = 4096
D = 3 * 32 * 32
NC, NS, L = 2, 16, 16
NW = NC * NS
ROWS_W = M // NW
NSPLIT = 5
SROWS = ROWS_W // NSPLIT
NGRP = (ROWS_W + L - 1) // L
LW = 25
LROWS = M // LW
BIG = 1 << 19
HUGE = 1 << 30
LSZ = NGRP * L + 2 * L


def _body(img_in, lbl_in, x_in, y_in, idx_in, img_out, lbl_out,
          winner_v, idx_v, y_v, lbl_v, shift_v, rows_l, src_l, gbuf,
          csems, gsem, osem):
    wid = lax.axis_index("s") * NC + lax.axis_index("c")
    row0 = wid * ROWS_W

    for j in range(NSPLIT):
        pltpu.async_copy(img_in.at[pl.ds(row0 + j * SROWS, SROWS)],
                         img_out.at[pl.ds(row0 + j * SROWS, SROWS)],
                         csems[j])

    pltpu.sync_copy(idx_in, idx_v)

    def init_body(i, c):
        winner_v[pl.ds(i * L, L)] = jnp.full((L,), -1, jnp.int32)
        return c
    lax.fori_loop(0, (M + L) // L, init_body, 0)

    shift_v[pl.ds(L, L)] = jnp.full((L,), HUGE, jnp.int32)
    lane = lax.iota(jnp.int32, L)

    def scan_body(v, c):
        vec = idx_v[pl.ds(v * L, L)]
        valid = vec < M
        key = jnp.where(valid, vec * L + lane, BIG + lane)
        skey, slane = plsc.sort_key_val(key, lane)
        shift_v[pl.ds(0, L)] = skey
        nkey = shift_v[pl.ds(1, L)]
        keep = ((skey >> 4) != (nkey >> 4)) & (skey < BIG)
        tgt = skey >> 4
        bvec = v * L + slane
        plsc.store_scatter(winner_v, [tgt], bvec, mask=keep)
        return c
    lax.fori_loop(0, B // L, scan_body, 0)

    @pl.when(wid < LW)
    def _labels():
        pltpu.sync_copy(y_in, y_v)
        l0 = wid * LROWS
        pltpu.sync_copy(lbl_in.at[pl.ds(l0, LROWS)], lbl_v)

        def lbl_body(v, c):
            wv = winner_v[pl.ds(l0 + v * L, L)]
            m = wv >= 0
            yv = plsc.load_gather(y_v, [jnp.maximum(wv, 0)])
            cur = lbl_v[pl.ds(v * L, L)]
            lbl_v[pl.ds(v * L, L)] = jnp.where(m, yv, cur)
            return c
        lax.fori_loop(0, LROWS // L, lbl_body, 0)
        pltpu.sync_copy(lbl_v, lbl_out.at[pl.ds(l0, LROWS)])

    def cmp_body(g, base):
        wv = winner_v[pl.ds(row0 + g * L, L)]
        rowv = row0 + g * L + lane
        m = (wv >= 0) & (g * L + lane < ROWS_W)
        pc = plsc.cumsum(jnp.where(m, 1, 0))
        pos = base + pc - 1
        plsc.store_scatter(rows_l, [pos], rowv, mask=m)
        plsc.store_scatter(src_l, [pos], wv, mask=m)
        return base + pc[L - 1]
    cnt = lax.fori_loop(0, NGRP, cmp_body, jnp.int32(0))

    @pl.when(cnt > 0)
    def _pad():
        lastrow = rows_l[pl.ds(cnt - 1, L)][0]
        lastsrc = src_l[pl.ds(cnt - 1, L)][0]
        rows_l[pl.ds(cnt, L)] = jnp.full((L,), lastrow, jnp.int32)
        src_l[pl.ds(cnt, L)] = jnp.full((L,), lastsrc, jnp.int32)

    for j in range(NSPLIT):
        pltpu.make_async_copy(img_in.at[pl.ds(row0 + j * SROWS, SROWS)],
                              img_out.at[pl.ds(row0 + j * SROWS, SROWS)],
                              csems[j]).wait()

    def patch_body(t, c):
        srcv = src_l[pl.ds(t * L, L)]
        rowv = rows_l[pl.ds(t * L, L)]
        pltpu.async_copy(x_in.at[srcv], gbuf, gsem).wait()
        pltpu.async_copy(gbuf, img_out.at[rowv], osem).wait()
        return c
    lax.fori_loop(0, (cnt + L - 1) // L, patch_body, 0)


@functools.cache
def _build():
    mesh = plsc.VectorSubcoreMesh(core_axis_name="c", subcore_axis_name="s",
                                  num_cores=NC, num_subcores=NS)
    return pl.kernel(
        _body,
        out_type=(jax.ShapeDtypeStruct((M, D), jnp.float32),
                  jax.ShapeDtypeStruct((M,), jnp.int32)),
        mesh=mesh,
        compiler_params=pltpu.CompilerParams(use_tc_tiling_on_sc=False,
                                             needs_layout_passes=False),
        scratch_types=dict(
            winner_v=pltpu.VMEM((M + L,), jnp.int32),
            idx_v=pltpu.VMEM((B,), jnp.int32),
            y_v=pltpu.VMEM((B,), jnp.int32),
            lbl_v=pltpu.VMEM((LROWS,), jnp.int32),
            shift_v=pltpu.VMEM((2 * L,), jnp.int32),
            rows_l=pltpu.VMEM((LSZ,), jnp.int32),
            src_l=pltpu.VMEM((LSZ,), jnp.int32),
            gbuf=pltpu.VMEM((L, D), jnp.float32),
            csems=[pltpu.SemaphoreType.DMA for _ in range(NSPLIT)],
            gsem=pltpu.SemaphoreType.DMA,
            osem=pltpu.SemaphoreType.DMA,
        ),
    )


def kernel(buffer_img, buffer_label, x, y, idx):
    img2 = buffer_img.reshape(M, D)
    x2 = x.reshape(B, D)
    out_img, out_lbl = _build()(img2, buffer_label, x2, y, idx)
    return out_img.reshape(buffer_img.shape), out_lbl


# speedup vs baseline: 7.9498x; 7.9498x over previous
import functools

import jax
import jax.numpy as jnp
from jax import lax
from jax.experimental import pallas as pl
from jax.experimental.pallas import tpu as pltpu
from jax.experimental.pallas import tpu_sc as plsc

M = 20000
B = 4096
D = 3 * 32 * 32
NC, NS, L = 2, 16, 16
NW = NC * NS
ROWS_W = M // NW
K = 5
NBUF = 5
NCH = ROWS_W // K
Q = 2
LW = 25
LROWS = M // LW
BIG = 1 << 19
HUGE = 1 << 30


def _body(img_in, lbl_in, x_in, y_in, idx_in, img_out, lbl_out,
          winner_v, idx_v, y_v, lbl_v, shift_v, bufs,
          in_sems, out_sems, xsem):
    wid = lax.axis_index("s") * NC + lax.axis_index("c")
    row0 = wid * ROWS_W

    pltpu.sync_copy(idx_in, idx_v)

    for b in range(NBUF):
        pltpu.async_copy(img_in.at[pl.ds(row0 + b * K, K)], bufs[b],
                         in_sems[b])

    def init_body(i, c):
        winner_v[pl.ds(i * L, L)] = jnp.full((L,), -1, jnp.int32)
        return c
    lax.fori_loop(0, (M + L) // L, init_body, 0)

    shift_v[pl.ds(L, L)] = jnp.full((L,), HUGE, jnp.int32)
    lane = lax.iota(jnp.int32, L)

    def scan_body(v, c):
        vec = idx_v[pl.ds(v * L, L)]
        valid = vec < M
        key = jnp.where(valid, vec * L + lane, BIG + lane)
        skey, slane = plsc.sort_key_val(key, lane)
        shift_v[pl.ds(0, L)] = skey
        nkey = shift_v[pl.ds(1, L)]
        keep = ((skey >> 4) != (nkey >> 4)) & (skey < BIG)
        tgt = skey >> 4
        bvec = v * L + slane
        plsc.store_scatter(winner_v, [tgt], bvec, mask=keep)
        return c
    lax.fori_loop(0, B // L, scan_body, 0)

    @pl.when(wid < LW)
    def _labels():
        pltpu.sync_copy(y_in, y_v)
        l0 = wid * LROWS
        pltpu.sync_copy(lbl_in.at[pl.ds(l0, LROWS)], lbl_v)

        def lbl_body(v, c):
            wv = winner_v[pl.ds(l0 + v * L, L)]
            m = wv >= 0
            yv = plsc.load_gather(y_v, [jnp.maximum(wv, 0)])
            cur = lbl_v[pl.ds(v * L, L)]
            lbl_v[pl.ds(v * L, L)] = jnp.where(m, yv, cur)
            return c
        lax.fori_loop(0, LROWS // L, lbl_body, 0)
        pltpu.sync_copy(lbl_v, lbl_out.at[pl.ds(l0, LROWS)])

    def outer(g, c):
        for b in range(NBUF):
            ch = g * NBUF + b
            r0 = row0 + ch * K
            pltpu.make_async_copy(img_in.at[pl.ds(r0, K)], bufs[b],
                                  in_sems[b]).wait()
            wvec = winner_v[pl.ds(r0, L)]
            for r in range(K):
                w = wvec[r]

                @pl.when(w >= 0)
                def _issue(w=w, b=b, r=r):
                    pltpu.async_copy(x_in.at[w], bufs[b].at[r], xsem)
            for r in range(K):
                w = wvec[r]

                @pl.when(w >= 0)
                def _drain(w=w, b=b, r=r):
                    pltpu.make_async_copy(x_in.at[w], bufs[b].at[r],
                                          xsem).wait()
            pltpu.async_copy(bufs[b], img_out.at[pl.ds(r0, K)], out_sems[b])

            bw = (b - Q) % NBUF
            ch_w = ch - Q

            @pl.when((ch >= Q) & (ch_w + NBUF < NCH))
            def _refill(bw=bw, ch_w=ch_w):
                r_w = row0 + ch_w * K
                pltpu.make_async_copy(bufs[bw], img_out.at[pl.ds(r_w, K)],
                                      out_sems[bw]).wait()
                r2 = row0 + (ch_w + NBUF) * K
                pltpu.async_copy(img_in.at[pl.ds(r2, K)], bufs[bw],
                                 in_sems[bw])
        return c
    lax.fori_loop(0, NCH // NBUF, outer, 0)

    for b in range(NBUF):
        ch_l = NCH - NBUF + b
        r_l = row0 + ch_l * K
        pltpu.make_async_copy(bufs[ch_l % NBUF], img_out.at[pl.ds(r_l, K)],
                              out_sems[ch_l % NBUF]).wait()


@functools.cache
def _build():
    mesh = plsc.VectorSubcoreMesh(core_axis_name="c", subcore_axis_name="s",
                                  num_cores=NC, num_subcores=NS)
    return pl.kernel(
        _body,
        out_type=(jax.ShapeDtypeStruct((M, D), jnp.float32),
                  jax.ShapeDtypeStruct((M,), jnp.int32)),
        mesh=mesh,
        compiler_params=pltpu.CompilerParams(use_tc_tiling_on_sc=False,
                                             needs_layout_passes=False),
        scratch_types=dict(
            winner_v=pltpu.VMEM((M + L,), jnp.int32),
            idx_v=pltpu.VMEM((B,), jnp.int32),
            y_v=pltpu.VMEM((B,), jnp.int32),
            lbl_v=pltpu.VMEM((LROWS,), jnp.int32),
            shift_v=pltpu.VMEM((2 * L,), jnp.int32),
            bufs=[pltpu.VMEM((K, D), jnp.float32) for _ in range(NBUF)],
            in_sems=[pltpu.SemaphoreType.DMA for _ in range(NBUF)],
            out_sems=[pltpu.SemaphoreType.DMA for _ in range(NBUF)],
            xsem=pltpu.SemaphoreType.DMA,
        ),
    )


def kernel(buffer_img, buffer_label, x, y, idx):
    img2 = buffer_img.reshape(M, D)
    x2 = x.reshape(B, D)
    out_img, out_lbl = _build()(img2, buffer_label, x2, y, idx)
    return out_img.reshape(buffer_img.shape), out_lbl
